# Initial kernel scaffold; baseline (speedup 1.0000x reference)
import jax
import jax.numpy as jnp
from jax.experimental import pallas as pl


def kernel(feats, coords, kernel, stride):
    raise NotImplementedError("write your pallas kernel here")


# speedup vs baseline: 1.2754x; 1.2754x over previous
import functools

import jax
import jax.numpy as jnp
from jax import lax
from jax.experimental import pallas as pl
from jax.experimental.pallas import tpu as pltpu
from jax.experimental.pallas import tpu_sc as plsc

N = 200000
IN_C = 32
OUT_C = 32
NK = 32
GRID = 32
NUM_SEG = GRID * GRID * GRID

EXT = 48
B1 = 1000
NW = 25
PPW = N // NW
SUB = 400
CH = 80
SEG_PER_TILE = NUM_SEG // 16


def _pass1_body(stride_ref, feats_ref, kidx_ref, k2_ref, ext_ref):
    f = feats_ref[...]
    p = jnp.dot(f, k2_ref[...], preferred_element_type=jnp.float32)
    kidx = kidx_ref[...] // stride_ref[0]
    acc = jnp.zeros((B1, OUT_C), jnp.float32)
    for k in range(NK):
        m = (kidx == k).astype(jnp.float32)
        acc = acc + p[:, k * OUT_C:(k + 1) * OUT_C] * m
    ones = jnp.ones((B1, 1), jnp.float32)
    pad = jnp.zeros((B1, EXT - OUT_C - 1), jnp.float32)
    ext_ref[...] = jnp.concatenate([acc, ones, pad], axis=1)


def _sc_scatter_body(ext_hbm, c0_hbm, c2_hbm, c3_hbm, part_hbm, rows_v, c0_v,
                     c2_v, c3_v, key_v, acc_sh):
    cid = lax.axis_index("c")
    sid = lax.axis_index("s")
    wid = sid * 2 + cid

    def zero_body(i, _):
        r = i // (EXT // 16)
        c = i % (EXT // 16)
        rows_v[r, pl.ds(pl.multiple_of(c * 16, 16), 16)] = jnp.zeros(
            (16,), jnp.float32)
        return 0
    lax.fori_loop(0, SUB * (EXT // 16), zero_body, 0)
    base_seg = sid * SEG_PER_TILE
    def zcp_body(m, _):
        pltpu.sync_copy(rows_v, acc_sh.at[pl.ds(base_seg + m * SUB, SUB)])
        return 0
    lax.fori_loop(0, SEG_PER_TILE // SUB, zcp_body, 0)
    rem = SEG_PER_TILE - (SEG_PER_TILE // SUB) * SUB
    if rem:
        pltpu.sync_copy(
            rows_v.at[pl.ds(0, rem)],
            acc_sh.at[pl.ds(base_seg + (SEG_PER_TILE // SUB) * SUB, rem)])
    plsc.subcore_barrier()

    @pl.when(wid < NW)
    def _scatter():
        base = wid * PPW

        def sub_body(j, _):
            off = pl.multiple_of(base + j * SUB, 8)
            pltpu.sync_copy(c0_hbm.at[pl.ds(off, SUB)], c0_v)
            pltpu.sync_copy(c2_hbm.at[pl.ds(off, SUB)], c2_v)
            pltpu.sync_copy(c3_hbm.at[pl.ds(off, SUB)], c3_v)

            def key_body(i, _):
                r = i // (CH // 16)
                v = i % (CH // 16)
                s = pl.ds(pl.multiple_of(r * CH + v * 16, 16), 16)
                key_v[r, pl.ds(pl.multiple_of(v * 16, 16), 16)] = (
                    (c0_v[s] * GRID + c2_v[s]) * GRID + c3_v[s])
                return 0
            lax.fori_loop(0, SUB // 16, key_body, 0)

            pltpu.sync_copy(ext_hbm.at[pl.ds(off, SUB)], rows_v)

            def sc_body(g, _):
                pltpu.sync_copy(
                    rows_v.at[pl.ds(pl.multiple_of(g * CH, 8), CH)],
                    acc_sh.at[key_v.at[g]], add=True)
                return 0
            lax.fori_loop(0, SUB // CH, sc_body, 0)
            return 0
        lax.fori_loop(0, PPW // SUB, sub_body, 0)

    plsc.subcore_barrier()
    pltpu.sync_copy(acc_sh.at[pl.ds(base_seg, SEG_PER_TILE)],
                    part_hbm.at[cid, pl.ds(base_seg, SEG_PER_TILE)])


S2 = 2048


def _pass2_body(part_ref, flat_ref, coords_ref):
    g = pl.program_id(0)
    s = part_ref[0] + part_ref[1]
    flat_ref[...] = s[:, :OUT_C]
    occ = s[:, OUT_C:OUT_C + 1] > 0.0
    row = g * S2 + lax.broadcasted_iota(jnp.int32, (S2, 4), 0)
    col = lax.broadcasted_iota(jnp.int32, (S2, 4), 1)
    vals = jnp.where(col == 0, row >> 10,
                     jnp.where(col == 2, (row >> 5) & (GRID - 1),
                               jnp.where(col == 3, row & (GRID - 1), 0)))
    coords_ref[...] = jnp.where(occ, vals, 0)


def kernel(feats, coords, kernel, stride):
    kern = kernel
    stride_arr = jnp.asarray(stride, jnp.int32).reshape(1)
    kidx2d = coords[:, 1:2]
    c0, c2, c3 = coords[:, 0], coords[:, 2], coords[:, 3]
    k2 = jnp.transpose(kern, (1, 0, 2)).reshape(IN_C, NK * OUT_C)

    ext = pl.pallas_call(
        _pass1_body,
        grid=(N // B1,),
        in_specs=[
            pl.BlockSpec(memory_space=pltpu.SMEM),
            pl.BlockSpec((B1, IN_C), lambda i: (i, 0)),
            pl.BlockSpec((B1, 1), lambda i: (i, 0)),
            pl.BlockSpec((IN_C, NK * OUT_C), lambda i: (0, 0)),
        ],
        out_specs=pl.BlockSpec((B1, EXT), lambda i: (i, 0)),
        out_shape=jax.ShapeDtypeStruct((N, EXT), jnp.float32),
    )(stride_arr, feats, kidx2d, k2)

    mesh = plsc.VectorSubcoreMesh(core_axis_name="c", subcore_axis_name="s")
    part = pl.kernel(
        _sc_scatter_body,
        out_type=jax.ShapeDtypeStruct((2, NUM_SEG, EXT), jnp.float32),
        mesh=mesh,
        compiler_params=pltpu.CompilerParams(use_tc_tiling_on_sc=False),
        scratch_types=[
            pltpu.VMEM((SUB, EXT), jnp.float32),
            pltpu.VMEM((SUB,), jnp.int32),
            pltpu.VMEM((SUB,), jnp.int32),
            pltpu.VMEM((SUB,), jnp.int32),
            pltpu.VMEM((SUB // CH, CH), jnp.int32),
            pltpu.VMEM_SHARED((NUM_SEG, EXT), jnp.float32),
        ],
    )(ext, c0, c2, c3)

    flat, out_coords = pl.pallas_call(
        _pass2_body,
        grid=(NUM_SEG // S2,),
        in_specs=[pl.BlockSpec((2, S2, EXT), lambda i: (0, i, 0))],
        out_specs=[
            pl.BlockSpec((S2, OUT_C), lambda i: (i, 0)),
            pl.BlockSpec((S2, 4), lambda i: (i, 0)),
        ],
        out_shape=[
            jax.ShapeDtypeStruct((NUM_SEG, OUT_C), jnp.float32),
            jax.ShapeDtypeStruct((NUM_SEG, 4), jnp.int32),
        ],
    )(part)

    return flat, out_coords


# speedup vs baseline: 2.1728x; 1.7037x over previous
import functools

import jax
import jax.numpy as jnp
from jax import lax
from jax.experimental import pallas as pl
from jax.experimental.pallas import tpu as pltpu
from jax.experimental.pallas import tpu_sc as plsc

N = 200000
IN_C = 32
OUT_C = 32
NK = 32
GRID = 32
NUM_SEG = GRID * GRID * GRID

EXT = 48
B1 = 1000
NW = 25
PPW = N // NW
SUB = 400
CH = 80
SEG_PER_TILE = NUM_SEG // 16


def _pass1_body(stride_ref, feats_ref, kidx_ref, kflat_ref, ext_ref):
    f = feats_ref[...]
    kidx = kidx_ref[...] // stride_ref[0]
    ftile = jnp.concatenate([f] * NK, axis=1)
    col_k = lax.broadcasted_iota(jnp.int32, (B1, NK * IN_C), 1) >> 5
    g = jnp.where(col_k == kidx, ftile, 0.0).astype(jnp.bfloat16)
    out = jnp.dot(g, kflat_ref[...], preferred_element_type=jnp.float32)
    ones = jnp.ones((B1, 1), jnp.float32)
    pad = jnp.zeros((B1, EXT - OUT_C - 1), jnp.float32)
    ext_ref[...] = jnp.concatenate([out, ones, pad], axis=1)


def _sc_scatter_body(ext_hbm, c0_hbm, c2_hbm, c3_hbm, part_hbm, rows_v, c0_v,
                     c2_v, c3_v, key_v, acc_sh):
    cid = lax.axis_index("c")
    sid = lax.axis_index("s")
    wid = sid * 2 + cid

    def zero_body(i, _):
        r = i // (EXT // 16)
        c = i % (EXT // 16)
        rows_v[r, pl.ds(pl.multiple_of(c * 16, 16), 16)] = jnp.zeros(
            (16,), jnp.float32)
        return 0
    lax.fori_loop(0, SUB * (EXT // 16), zero_body, 0)
    base_seg = sid * SEG_PER_TILE
    def zcp_body(m, _):
        pltpu.sync_copy(rows_v, acc_sh.at[pl.ds(base_seg + m * SUB, SUB)])
        return 0
    lax.fori_loop(0, SEG_PER_TILE // SUB, zcp_body, 0)
    rem = SEG_PER_TILE - (SEG_PER_TILE // SUB) * SUB
    if rem:
        pltpu.sync_copy(
            rows_v.at[pl.ds(0, rem)],
            acc_sh.at[pl.ds(base_seg + (SEG_PER_TILE // SUB) * SUB, rem)])
    plsc.subcore_barrier()

    @pl.when(wid < NW)
    def _scatter():
        base = wid * PPW

        def sub_body(j, _):
            off = pl.multiple_of(base + j * SUB, 8)
            pltpu.sync_copy(c0_hbm.at[pl.ds(off, SUB)], c0_v)
            pltpu.sync_copy(c2_hbm.at[pl.ds(off, SUB)], c2_v)
            pltpu.sync_copy(c3_hbm.at[pl.ds(off, SUB)], c3_v)

            def key_body(i, _):
                r = i // (CH // 16)
                v = i % (CH // 16)
                s = pl.ds(pl.multiple_of(r * CH + v * 16, 16), 16)
                key_v[r, pl.ds(pl.multiple_of(v * 16, 16), 16)] = (
                    (c0_v[s] * GRID + c2_v[s]) * GRID + c3_v[s])
                return 0
            lax.fori_loop(0, SUB // 16, key_body, 0)

            pltpu.sync_copy(ext_hbm.at[pl.ds(off, SUB)], rows_v)

            def sc_body(g, _):
                pltpu.sync_copy(
                    rows_v.at[pl.ds(pl.multiple_of(g * CH, 8), CH)],
                    acc_sh.at[key_v.at[g]], add=True)
                return 0
            lax.fori_loop(0, SUB // CH, sc_body, 0)
            return 0
        lax.fori_loop(0, PPW // SUB, sub_body, 0)

    plsc.subcore_barrier()
    pltpu.sync_copy(acc_sh.at[pl.ds(base_seg, SEG_PER_TILE)],
                    part_hbm.at[cid, pl.ds(base_seg, SEG_PER_TILE)])


S2 = 2048


def _pass2_body(part_ref, flat_ref, coords_ref):
    g = pl.program_id(0)
    s = part_ref[0] + part_ref[1]
    flat_ref[...] = s[:, :OUT_C]
    occ = s[:, OUT_C:OUT_C + 1] > 0.0
    row = g * S2 + lax.broadcasted_iota(jnp.int32, (S2, 4), 0)
    col = lax.broadcasted_iota(jnp.int32, (S2, 4), 1)
    vals = jnp.where(col == 0, row >> 10,
                     jnp.where(col == 2, (row >> 5) & (GRID - 1),
                               jnp.where(col == 3, row & (GRID - 1), 0)))
    coords_ref[...] = jnp.where(occ, vals, 0)


def kernel(feats, coords, kernel, stride):
    kern = kernel
    stride_arr = jnp.asarray(stride, jnp.int32).reshape(1)
    kidx2d = coords[:, 1:2]
    c0, c2, c3 = coords[:, 0], coords[:, 2], coords[:, 3]
    kflat = kern.reshape(NK * IN_C, OUT_C).astype(jnp.bfloat16)

    ext = pl.pallas_call(
        _pass1_body,
        grid=(N // B1,),
        in_specs=[
            pl.BlockSpec(memory_space=pltpu.SMEM),
            pl.BlockSpec((B1, IN_C), lambda i: (i, 0)),
            pl.BlockSpec((B1, 1), lambda i: (i, 0)),
            pl.BlockSpec((NK * IN_C, OUT_C), lambda i: (0, 0)),
        ],
        out_specs=pl.BlockSpec((B1, EXT), lambda i: (i, 0)),
        out_shape=jax.ShapeDtypeStruct((N, EXT), jnp.float32),
    )(stride_arr, feats, kidx2d, kflat)

    mesh = plsc.VectorSubcoreMesh(core_axis_name="c", subcore_axis_name="s")
    part = pl.kernel(
        _sc_scatter_body,
        out_type=jax.ShapeDtypeStruct((2, NUM_SEG, EXT), jnp.float32),
        mesh=mesh,
        compiler_params=pltpu.CompilerParams(use_tc_tiling_on_sc=False),
        scratch_types=[
            pltpu.VMEM((SUB, EXT), jnp.float32),
            pltpu.VMEM((SUB,), jnp.int32),
            pltpu.VMEM((SUB,), jnp.int32),
            pltpu.VMEM((SUB,), jnp.int32),
            pltpu.VMEM((SUB // CH, CH), jnp.int32),
            pltpu.VMEM_SHARED((NUM_SEG, EXT), jnp.float32),
        ],
    )(ext, c0, c2, c3)

    flat, out_coords = pl.pallas_call(
        _pass2_body,
        grid=(NUM_SEG // S2,),
        in_specs=[pl.BlockSpec((2, S2, EXT), lambda i: (0, i, 0))],
        out_specs=[
            pl.BlockSpec((S2, OUT_C), lambda i: (i, 0)),
            pl.BlockSpec((S2, 4), lambda i: (i, 0)),
        ],
        out_shape=[
            jax.ShapeDtypeStruct((NUM_SEG, OUT_C), jnp.float32),
            jax.ShapeDtypeStruct((NUM_SEG, 4), jnp.int32),
        ],
    )(part)

    return flat, out_coords


# speedup vs baseline: 5.2461x; 2.4144x over previous
import functools

import jax
import jax.numpy as jnp
from jax import lax
from jax.experimental import pallas as pl
from jax.experimental.pallas import tpu as pltpu
from jax.experimental.pallas import tpu_sc as plsc

N = 200000
IN_C = 32
OUT_C = 32
NK = 32
GRID = 32
NUM_SEG = GRID * GRID * GRID

EXT = 48
B1 = 1000
NW = 25
PPW = N // NW
SUB = 400
CH = 80
SEG_PER_TILE = NUM_SEG // 16


def _pass1_body(mdiv_ref, feats_ref, kidx_ref, w_ref, ext_ref):
    f = feats_ref[...]
    kidx = (kidx_ref[...] * mdiv_ref[0]) >> 10
    lo = kidx & 7
    hi = kidx >> 3
    ftile = jnp.concatenate([f] * 8, axis=1)
    col_lo = lax.broadcasted_iota(jnp.int32, (B1, 8 * IN_C), 1) >> 5
    g1 = jnp.where(col_lo == lo, ftile, 0.0).astype(jnp.bfloat16)
    ph = jnp.dot(g1, w_ref[...], preferred_element_type=jnp.float32)
    col_hi = lax.broadcasted_iota(jnp.int32, (B1, 4 * OUT_C), 1) >> 5
    m = jnp.where(col_hi == hi, ph, 0.0)
    out = ((m[:, :OUT_C] + m[:, OUT_C:2 * OUT_C])
           + (m[:, 2 * OUT_C:3 * OUT_C] + m[:, 3 * OUT_C:]))
    ones = jnp.ones((B1, 1), jnp.float32)
    pad = jnp.zeros((B1, EXT - OUT_C - 1), jnp.float32)
    ext_ref[...] = jnp.concatenate([out, ones, pad], axis=1)


def _sc_scatter_body(ext_hbm, c0_hbm, c2_hbm, c3_hbm, part_hbm, rows_v, c0_v,
                     c2_v, c3_v, key_v, acc_sh):
    cid = lax.axis_index("c")
    sid = lax.axis_index("s")
    wid = sid * 2 + cid

    def zero_body(i, _):
        r = i // (EXT // 16)
        c = i % (EXT // 16)
        rows_v[r, pl.ds(pl.multiple_of(c * 16, 16), 16)] = jnp.zeros(
            (16,), jnp.float32)
        return 0
    lax.fori_loop(0, SUB * (EXT // 16), zero_body, 0)
    base_seg = sid * SEG_PER_TILE
    def zcp_body(m, _):
        pltpu.sync_copy(rows_v, acc_sh.at[pl.ds(base_seg + m * SUB, SUB)])
        return 0
    lax.fori_loop(0, SEG_PER_TILE // SUB, zcp_body, 0)
    rem = SEG_PER_TILE - (SEG_PER_TILE // SUB) * SUB
    if rem:
        pltpu.sync_copy(
            rows_v.at[pl.ds(0, rem)],
            acc_sh.at[pl.ds(base_seg + (SEG_PER_TILE // SUB) * SUB, rem)])
    plsc.subcore_barrier()

    @pl.when(wid < NW)
    def _scatter():
        base = wid * PPW

        def sub_body(j, _):
            off = pl.multiple_of(base + j * SUB, 8)
            pltpu.sync_copy(c0_hbm.at[pl.ds(off, SUB)], c0_v)
            pltpu.sync_copy(c2_hbm.at[pl.ds(off, SUB)], c2_v)
            pltpu.sync_copy(c3_hbm.at[pl.ds(off, SUB)], c3_v)

            def key_body(i, _):
                r = i // (CH // 16)
                v = i % (CH // 16)
                s = pl.ds(pl.multiple_of(r * CH + v * 16, 16), 16)
                key_v[r, pl.ds(pl.multiple_of(v * 16, 16), 16)] = (
                    (c0_v[s] * GRID + c2_v[s]) * GRID + c3_v[s])
                return 0
            lax.fori_loop(0, SUB // 16, key_body, 0)

            pltpu.sync_copy(ext_hbm.at[pl.ds(off, SUB)], rows_v)

            def sc_body(g, _):
                pltpu.sync_copy(
                    rows_v.at[pl.ds(pl.multiple_of(g * CH, 8), CH)],
                    acc_sh.at[key_v.at[g]], add=True)
                return 0
            lax.fori_loop(0, SUB // CH, sc_body, 0)
            return 0
        lax.fori_loop(0, PPW // SUB, sub_body, 0)

    plsc.subcore_barrier()
    pltpu.sync_copy(acc_sh.at[pl.ds(base_seg, SEG_PER_TILE)],
                    part_hbm.at[cid, pl.ds(base_seg, SEG_PER_TILE)])


S2 = 2048


def _pass2_body(part_ref, flat_ref, coords_ref):
    g = pl.program_id(0)
    s = part_ref[0] + part_ref[1]
    flat_ref[...] = s[:, :OUT_C]
    occ = s[:, OUT_C:OUT_C + 1] > 0.0
    row = g * S2 + lax.broadcasted_iota(jnp.int32, (S2, 4), 0)
    col = lax.broadcasted_iota(jnp.int32, (S2, 4), 1)
    vals = jnp.where(col == 0, row >> 10,
                     jnp.where(col == 2, (row >> 5) & (GRID - 1),
                               jnp.where(col == 3, row & (GRID - 1), 0)))
    coords_ref[...] = jnp.where(occ, vals, 0)


def kernel(feats, coords, kernel, stride):
    kern = kernel
    stride_i = jnp.asarray(stride, jnp.int32)
    mdiv = ((1024 + stride_i - 1) // stride_i).reshape(1)
    kidx2d = coords[:, 1:2]
    c0, c2, c3 = coords[:, 0], coords[:, 2], coords[:, 3]
    w = (kern.reshape(4, 8, IN_C, OUT_C).transpose(1, 2, 0, 3)
         .reshape(8 * IN_C, 4 * OUT_C).astype(jnp.bfloat16))

    ext = pl.pallas_call(
        _pass1_body,
        grid=(N // B1,),
        in_specs=[
            pl.BlockSpec(memory_space=pltpu.SMEM),
            pl.BlockSpec((B1, IN_C), lambda i: (i, 0)),
            pl.BlockSpec((B1, 1), lambda i: (i, 0)),
            pl.BlockSpec((8 * IN_C, 4 * OUT_C), lambda i: (0, 0)),
        ],
        out_specs=pl.BlockSpec((B1, EXT), lambda i: (i, 0)),
        out_shape=jax.ShapeDtypeStruct((N, EXT), jnp.float32),
    )(mdiv, feats, kidx2d, w)

    mesh = plsc.VectorSubcoreMesh(core_axis_name="c", subcore_axis_name="s")
    part = pl.kernel(
        _sc_scatter_body,
        out_type=jax.ShapeDtypeStruct((2, NUM_SEG, EXT), jnp.float32),
        mesh=mesh,
        compiler_params=pltpu.CompilerParams(use_tc_tiling_on_sc=False),
        scratch_types=[
            pltpu.VMEM((SUB, EXT), jnp.float32),
            pltpu.VMEM((SUB,), jnp.int32),
            pltpu.VMEM((SUB,), jnp.int32),
            pltpu.VMEM((SUB,), jnp.int32),
            pltpu.VMEM((SUB // CH, CH), jnp.int32),
            pltpu.VMEM_SHARED((NUM_SEG, EXT), jnp.float32),
        ],
    )(ext, c0, c2, c3)

    flat, out_coords = pl.pallas_call(
        _pass2_body,
        grid=(NUM_SEG // S2,),
        in_specs=[pl.BlockSpec((2, S2, EXT), lambda i: (0, i, 0))],
        out_specs=[
            pl.BlockSpec((S2, OUT_C), lambda i: (i, 0)),
            pl.BlockSpec((S2, 4), lambda i: (i, 0)),
        ],
        out_shape=[
            jax.ShapeDtypeStruct((NUM_SEG, OUT_C), jnp.float32),
            jax.ShapeDtypeStruct((NUM_SEG, 4), jnp.int32),
        ],
    )(part)

    return flat, out_coords


# speedup vs baseline: 5.2720x; 1.0049x over previous
import jax
import jax.numpy as jnp
from jax import lax
from jax.experimental import pallas as pl
from jax.experimental.pallas import tpu as pltpu
from jax.experimental.pallas import tpu_sc as plsc

N = 200000
IN_C = 32
OUT_C = 32
NK = 32
GRID = 32
NUM_SEG = GRID * GRID * GRID

EXT = 48
B1 = 1000
NW = 25
PPW = N // NW
SUB = 400
CH = 80
SEG_PER_TILE = NUM_SEG // 16


def _pass1_body(mdiv_ref, feats_ref, coords_ref, w_ref, ext_ref):
    f = feats_ref[...]
    c = coords_ref[...]
    kidx = (c[:, 1:2] * mdiv_ref[0]) >> 10
    key = (c[:, 0:1] * GRID + c[:, 2:3]) * GRID + c[:, 3:4]
    keyf = lax.bitcast_convert_type(key, jnp.float32)
    lo = kidx & 7
    hi = kidx >> 3
    ftile = jnp.concatenate([f] * 8, axis=1)
    col_lo = lax.broadcasted_iota(jnp.int32, (B1, 8 * IN_C), 1) >> 5
    g1 = jnp.where(col_lo == lo, ftile, 0.0).astype(jnp.bfloat16)
    ph = jnp.dot(g1, w_ref[...], preferred_element_type=jnp.float32)
    col_hi = lax.broadcasted_iota(jnp.int32, (B1, 4 * OUT_C), 1) >> 5
    m = jnp.where(col_hi == hi, ph, 0.0)
    out = ((m[:, :OUT_C] + m[:, OUT_C:2 * OUT_C])
           + (m[:, 2 * OUT_C:3 * OUT_C] + m[:, 3 * OUT_C:]))
    ones = jnp.ones((B1, 1), jnp.float32)
    pad = jnp.zeros((B1, EXT - OUT_C - 2), jnp.float32)
    ext_ref[...] = jnp.concatenate([out, ones, keyf, pad], axis=1)


def _sc_scatter_body(ext_hbm, part_hbm, rows_v, key_v, acc_sh):
    cid = lax.axis_index("c")
    sid = lax.axis_index("s")
    wid = sid * 2 + cid

    def zero_body(i, _):
        rows_v[i // (EXT // 16),
               pl.ds(pl.multiple_of((i % (EXT // 16)) * 16, 16), 16)] = (
            jnp.zeros((16,), jnp.float32))
        return 0
    lax.fori_loop(0, SUB * (EXT // 16), zero_body, 0)
    base_seg = sid * SEG_PER_TILE
    def zcp_body(m, _):
        pltpu.sync_copy(rows_v, acc_sh.at[pl.ds(base_seg + m * SUB, SUB)])
        return 0
    lax.fori_loop(0, SEG_PER_TILE // SUB, zcp_body, 0)
    rem = SEG_PER_TILE - (SEG_PER_TILE // SUB) * SUB
    if rem:
        pltpu.sync_copy(
            rows_v.at[pl.ds(0, rem)],
            acc_sh.at[pl.ds(base_seg + (SEG_PER_TILE // SUB) * SUB, rem)])
    plsc.subcore_barrier()

    @pl.when(wid < NW)
    def _scatter():
        base = wid * PPW
        lane_key = jnp.full((16,), OUT_C + 1, jnp.int32)

        def sub_body(j, _):
            off = pl.multiple_of(base + j * SUB, 8)
            pltpu.sync_copy(ext_hbm.at[pl.ds(off, SUB)], rows_v)

            def key_body(i, _):
                ridx = i * 16 + lax.broadcasted_iota(jnp.int32, (16,), 0)
                keyf = plsc.load_gather(rows_v, [ridx, lane_key])
                key_v[i // (CH // 16),
                      pl.ds(pl.multiple_of((i % (CH // 16)) * 16, 16), 16)] = (
                    plsc.bitcast(keyf, jnp.int32))
                return 0
            lax.fori_loop(0, SUB // 16, key_body, 0)

            def sc_body(g, _):
                pltpu.sync_copy(
                    rows_v.at[pl.ds(pl.multiple_of(g * CH, 8), CH)],
                    acc_sh.at[key_v.at[g]], add=True)
                return 0
            lax.fori_loop(0, SUB // CH, sc_body, 0)
            return 0
        lax.fori_loop(0, PPW // SUB, sub_body, 0)

    plsc.subcore_barrier()
    pltpu.sync_copy(acc_sh.at[pl.ds(base_seg, SEG_PER_TILE)],
                    part_hbm.at[cid, pl.ds(base_seg, SEG_PER_TILE)])


S2 = 2048


def _pass2_body(part_ref, flat_ref, coords_ref):
    g = pl.program_id(0)
    s = part_ref[0] + part_ref[1]
    flat_ref[...] = s[:, :OUT_C]
    occ = s[:, OUT_C:OUT_C + 1] > 0.0
    row = g * S2 + lax.broadcasted_iota(jnp.int32, (S2, 4), 0)
    col = lax.broadcasted_iota(jnp.int32, (S2, 4), 1)
    vals = jnp.where(col == 0, row >> 10,
                     jnp.where(col == 2, (row >> 5) & (GRID - 1),
                               jnp.where(col == 3, row & (GRID - 1), 0)))
    coords_ref[...] = jnp.where(occ, vals, 0)


def kernel(feats, coords, kernel, stride):
    kern = kernel
    stride_i = jnp.asarray(stride, jnp.int32)
    mdiv = ((1024 + stride_i - 1) // stride_i).reshape(1)
    w = (kern.reshape(4, 8, IN_C, OUT_C).transpose(1, 2, 0, 3)
         .reshape(8 * IN_C, 4 * OUT_C).astype(jnp.bfloat16))

    ext = pl.pallas_call(
        _pass1_body,
        grid=(N // B1,),
        in_specs=[
            pl.BlockSpec(memory_space=pltpu.SMEM),
            pl.BlockSpec((B1, IN_C), lambda i: (i, 0)),
            pl.BlockSpec((B1, 4), lambda i: (i, 0)),
            pl.BlockSpec((8 * IN_C, 4 * OUT_C), lambda i: (0, 0)),
        ],
        out_specs=pl.BlockSpec((B1, EXT), lambda i: (i, 0)),
        out_shape=jax.ShapeDtypeStruct((N, EXT), jnp.float32),
    )(mdiv, feats, coords, w)

    mesh = plsc.VectorSubcoreMesh(core_axis_name="c", subcore_axis_name="s")
    part = pl.kernel(
        _sc_scatter_body,
        out_type=jax.ShapeDtypeStruct((2, NUM_SEG, EXT), jnp.float32),
        mesh=mesh,
        compiler_params=pltpu.CompilerParams(
            use_tc_tiling_on_sc=False, needs_layout_passes=False),
        scratch_types=[
            pltpu.VMEM((SUB, EXT), jnp.float32),
            pltpu.VMEM((SUB // CH, CH), jnp.int32),
            pltpu.VMEM_SHARED((NUM_SEG, EXT), jnp.float32),
        ],
    )(ext)

    flat, out_coords = pl.pallas_call(
        _pass2_body,
        grid=(NUM_SEG // S2,),
        in_specs=[pl.BlockSpec((2, S2, EXT), lambda i: (0, i, 0))],
        out_specs=[
            pl.BlockSpec((S2, OUT_C), lambda i: (i, 0)),
            pl.BlockSpec((S2, 4), lambda i: (i, 0)),
        ],
        out_shape=[
            jax.ShapeDtypeStruct((NUM_SEG, OUT_C), jnp.float32),
            jax.ShapeDtypeStruct((NUM_SEG, 4), jnp.int32),
        ],
    )(part)

    return flat, out_coords


# speedup vs baseline: 5.6220x; 1.0664x over previous
import jax
import jax.numpy as jnp
from jax import lax
from jax.experimental import pallas as pl
from jax.experimental.pallas import tpu as pltpu
from jax.experimental.pallas import tpu_sc as plsc

N = 200000
IN_C = 32
OUT_C = 32
NK = 32
GRID = 32
NUM_SEG = GRID * GRID * GRID

EXT = 48
B1 = 2000
NW = 25
PPW = N // NW
SUB = 400
CH = 80
SEG_PER_TILE = NUM_SEG // 16


def _pass1_body(mdiv_ref, feats_ref, coords_ref, w_ref, ext_ref):
    f = feats_ref[...]
    c = coords_ref[...]
    kidx = (c[:, 1:2] * mdiv_ref[0]) >> 10
    key = (c[:, 0:1] * GRID + c[:, 2:3]) * GRID + c[:, 3:4]
    keyf = lax.bitcast_convert_type(key, jnp.float32)
    lo = kidx & 7
    hi = kidx >> 3
    ftile = jnp.concatenate([f] * 8, axis=1)
    col_lo = lax.broadcasted_iota(jnp.int32, (B1, 8 * IN_C), 1) >> 5
    g1 = jnp.where(col_lo == lo, ftile, 0.0).astype(jnp.bfloat16)
    ph = jnp.dot(g1, w_ref[...], preferred_element_type=jnp.float32)
    col_hi = lax.broadcasted_iota(jnp.int32, (B1, 4 * OUT_C), 1) >> 5
    m = jnp.where(col_hi == hi, ph, 0.0)
    out = ((m[:, :OUT_C] + m[:, OUT_C:2 * OUT_C])
           + (m[:, 2 * OUT_C:3 * OUT_C] + m[:, 3 * OUT_C:]))
    ones = jnp.ones((B1, 1), jnp.float32)
    pad = jnp.zeros((B1, EXT - OUT_C - 2), jnp.float32)
    ext_ref[...] = jnp.concatenate([out, ones, keyf, pad], axis=1)


def _sc_scatter_body(ext_hbm, part_hbm, rows_v, key_v, acc_sh):
    cid = lax.axis_index("c")
    sid = lax.axis_index("s")
    wid = sid * 2 + cid

    def zero_body(i, _):
        rows_v[i // (EXT // 16),
               pl.ds(pl.multiple_of((i % (EXT // 16)) * 16, 16), 16)] = (
            jnp.zeros((16,), jnp.float32))
        return 0
    lax.fori_loop(0, SUB * (EXT // 16), zero_body, 0)
    base_seg = sid * SEG_PER_TILE
    def zcp_body(m, _):
        pltpu.sync_copy(rows_v, acc_sh.at[pl.ds(base_seg + m * SUB, SUB)])
        return 0
    lax.fori_loop(0, SEG_PER_TILE // SUB, zcp_body, 0)
    rem = SEG_PER_TILE - (SEG_PER_TILE // SUB) * SUB
    if rem:
        pltpu.sync_copy(
            rows_v.at[pl.ds(0, rem)],
            acc_sh.at[pl.ds(base_seg + (SEG_PER_TILE // SUB) * SUB, rem)])
    plsc.subcore_barrier()

    @pl.when(wid < NW)
    def _scatter():
        base = wid * PPW
        lane_key = jnp.full((16,), OUT_C + 1, jnp.int32)

        def sub_body(j, _):
            off = pl.multiple_of(base + j * SUB, 8)
            pltpu.sync_copy(ext_hbm.at[pl.ds(off, SUB)], rows_v)

            def key_body(i, _):
                ridx = i * 16 + lax.broadcasted_iota(jnp.int32, (16,), 0)
                keyf = plsc.load_gather(rows_v, [ridx, lane_key])
                key_v[i // (CH // 16),
                      pl.ds(pl.multiple_of((i % (CH // 16)) * 16, 16), 16)] = (
                    plsc.bitcast(keyf, jnp.int32))
                return 0
            lax.fori_loop(0, SUB // 16, key_body, 0)

            def sc_body(g, _):
                pltpu.sync_copy(
                    rows_v.at[pl.ds(pl.multiple_of(g * CH, 8), CH)],
                    acc_sh.at[key_v.at[g]], add=True)
                return 0
            lax.fori_loop(0, SUB // CH, sc_body, 0)
            return 0
        lax.fori_loop(0, PPW // SUB, sub_body, 0)

    plsc.subcore_barrier()
    pltpu.sync_copy(acc_sh.at[pl.ds(base_seg, SEG_PER_TILE)],
                    part_hbm.at[cid, pl.ds(base_seg, SEG_PER_TILE)])


S2 = 2048


def _pass2_body(part_ref, flat_ref, coords_ref):
    g = pl.program_id(0)
    s = part_ref[0] + part_ref[1]
    flat_ref[...] = s[:, :OUT_C]
    occ = s[:, OUT_C:OUT_C + 1] > 0.0
    row = g * S2 + lax.broadcasted_iota(jnp.int32, (S2, 4), 0)
    col = lax.broadcasted_iota(jnp.int32, (S2, 4), 1)
    vals = jnp.where(col == 0, row >> 10,
                     jnp.where(col == 2, (row >> 5) & (GRID - 1),
                               jnp.where(col == 3, row & (GRID - 1), 0)))
    coords_ref[...] = jnp.where(occ, vals, 0)


def kernel(feats, coords, kernel, stride):
    kern = kernel
    stride_i = jnp.asarray(stride, jnp.int32)
    mdiv = ((1024 + stride_i - 1) // stride_i).reshape(1)
    w = (kern.reshape(4, 8, IN_C, OUT_C).transpose(1, 2, 0, 3)
         .reshape(8 * IN_C, 4 * OUT_C).astype(jnp.bfloat16))

    ext = pl.pallas_call(
        _pass1_body,
        grid=(N // B1,),
        in_specs=[
            pl.BlockSpec(memory_space=pltpu.SMEM),
            pl.BlockSpec((B1, IN_C), lambda i: (i, 0)),
            pl.BlockSpec((B1, 4), lambda i: (i, 0)),
            pl.BlockSpec((8 * IN_C, 4 * OUT_C), lambda i: (0, 0)),
        ],
        out_specs=pl.BlockSpec((B1, EXT), lambda i: (i, 0)),
        out_shape=jax.ShapeDtypeStruct((N, EXT), jnp.float32),
    )(mdiv, feats, coords, w)

    mesh = plsc.VectorSubcoreMesh(core_axis_name="c", subcore_axis_name="s")
    part = pl.kernel(
        _sc_scatter_body,
        out_type=jax.ShapeDtypeStruct((2, NUM_SEG, EXT), jnp.float32),
        mesh=mesh,
        compiler_params=pltpu.CompilerParams(
            use_tc_tiling_on_sc=False, needs_layout_passes=False),
        scratch_types=[
            pltpu.VMEM((SUB, EXT), jnp.float32),
            pltpu.VMEM((SUB // CH, CH), jnp.int32),
            pltpu.VMEM_SHARED((NUM_SEG, EXT), jnp.float32),
        ],
    )(ext)

    flat, out_coords = pl.pallas_call(
        _pass2_body,
        grid=(NUM_SEG // S2,),
        in_specs=[pl.BlockSpec((2, S2, EXT), lambda i: (0, i, 0))],
        out_specs=[
            pl.BlockSpec((S2, OUT_C), lambda i: (i, 0)),
            pl.BlockSpec((S2, 4), lambda i: (i, 0)),
        ],
        out_shape=[
            jax.ShapeDtypeStruct((NUM_SEG, OUT_C), jnp.float32),
            jax.ShapeDtypeStruct((NUM_SEG, 4), jnp.int32),
        ],
    )(part)

    return flat, out_coords


# speedup vs baseline: 9.5427x; 1.6974x over previous
import jax
import jax.numpy as jnp
from jax import lax
from jax.experimental import pallas as pl
from jax.experimental.pallas import tpu as pltpu
from jax.experimental.pallas import tpu_sc as plsc

N = 200000
IN_C = 32
OUT_C = 32
NK = 32
GRID = 32
NUM_SEG = GRID * GRID * GRID

EXT = 48
B1 = 2048
NW = 25
PPW = N // NW
SUB = 400
CH = 80
SEG_PER_TILE = NUM_SEG // 16


def _pass1_body(mdiv_ref, ft_ref, ct_ref, wt_ref, ext_ref):
    ft = ft_ref[...]
    ct = ct_ref[...]
    kidx = (ct[1:2, :] * mdiv_ref[0]) >> 10
    key = (ct[0:1, :] * GRID + ct[2:3, :]) * GRID + ct[3:4, :]
    keyf = lax.bitcast_convert_type(key, jnp.float32)
    lo = kidx & 7
    hi = kidx >> 3
    gt = jnp.concatenate([ft] * 8, axis=0)
    row_lo = lax.broadcasted_iota(jnp.int32, (8 * IN_C, B1), 0) >> 5
    g1t = jnp.where(row_lo == lo, gt, 0.0).astype(jnp.bfloat16)
    pht = lax.dot_general(wt_ref[...], g1t, (((1,), (0,)), ((), ())),
                          preferred_element_type=jnp.float32)
    row_hi = lax.broadcasted_iota(jnp.int32, (4 * OUT_C, B1), 0) >> 5
    mt = jnp.where(row_hi == hi, pht, 0.0)
    outt = ((mt[:OUT_C, :] + mt[OUT_C:2 * OUT_C, :])
            + (mt[2 * OUT_C:3 * OUT_C, :] + mt[3 * OUT_C:, :]))
    ones = jnp.ones((1, B1), jnp.float32)
    pad = jnp.zeros((EXT - OUT_C - 2, B1), jnp.float32)
    extt = jnp.concatenate([outt, ones, keyf, pad], axis=0)
    ext_ref[...] = extt.T


def _sc_scatter_body(ext_hbm, part_hbm, rows_v, key_v, acc_sh):
    cid = lax.axis_index("c")
    sid = lax.axis_index("s")
    wid = sid * 2 + cid

    def zero_body(i, _):
        rows_v[i // (EXT // 16),
               pl.ds(pl.multiple_of((i % (EXT // 16)) * 16, 16), 16)] = (
            jnp.zeros((16,), jnp.float32))
        return 0
    lax.fori_loop(0, SUB * (EXT // 16), zero_body, 0)
    base_seg = sid * SEG_PER_TILE
    def zcp_body(m, _):
        pltpu.sync_copy(rows_v, acc_sh.at[pl.ds(base_seg + m * SUB, SUB)])
        return 0
    lax.fori_loop(0, SEG_PER_TILE // SUB, zcp_body, 0)
    rem = SEG_PER_TILE - (SEG_PER_TILE // SUB) * SUB
    if rem:
        pltpu.sync_copy(
            rows_v.at[pl.ds(0, rem)],
            acc_sh.at[pl.ds(base_seg + (SEG_PER_TILE // SUB) * SUB, rem)])
    plsc.subcore_barrier()

    @pl.when(wid < NW)
    def _scatter():
        base = wid * PPW
        lane_key = jnp.full((16,), OUT_C + 1, jnp.int32)

        def sub_body(j, _):
            off = pl.multiple_of(base + j * SUB, 8)
            pltpu.sync_copy(ext_hbm.at[pl.ds(off, SUB)], rows_v)

            def key_body(i, _):
                ridx = i * 16 + lax.broadcasted_iota(jnp.int32, (16,), 0)
                keyf = plsc.load_gather(rows_v, [ridx, lane_key])
                key_v[i // (CH // 16),
                      pl.ds(pl.multiple_of((i % (CH // 16)) * 16, 16), 16)] = (
                    plsc.bitcast(keyf, jnp.int32))
                return 0
            lax.fori_loop(0, SUB // 16, key_body, 0)

            def sc_body(g, _):
                pltpu.sync_copy(
                    rows_v.at[pl.ds(pl.multiple_of(g * CH, 8), CH)],
                    acc_sh.at[key_v.at[g]], add=True)
                return 0
            lax.fori_loop(0, SUB // CH, sc_body, 0)
            return 0
        lax.fori_loop(0, PPW // SUB, sub_body, 0)

    plsc.subcore_barrier()
    pltpu.sync_copy(acc_sh.at[pl.ds(base_seg, SEG_PER_TILE)],
                    part_hbm.at[cid, pl.ds(base_seg, SEG_PER_TILE)])


S2 = 2048


def _pass2_body(part_ref, flat_ref, coords_ref):
    g = pl.program_id(0)
    s = part_ref[0] + part_ref[1]
    flat_ref[...] = s[:, :OUT_C]
    occ = s[:, OUT_C:OUT_C + 1] > 0.0
    row = g * S2 + lax.broadcasted_iota(jnp.int32, (S2, 4), 0)
    col = lax.broadcasted_iota(jnp.int32, (S2, 4), 1)
    vals = jnp.where(col == 0, row >> 10,
                     jnp.where(col == 2, (row >> 5) & (GRID - 1),
                               jnp.where(col == 3, row & (GRID - 1), 0)))
    coords_ref[...] = jnp.where(occ, vals, 0)


def kernel(feats, coords, kernel, stride):
    kern = kernel
    stride_i = jnp.asarray(stride, jnp.int32)
    mdiv = ((1024 + stride_i - 1) // stride_i).reshape(1)
    wt = (kern.reshape(4, 8, IN_C, OUT_C).transpose(0, 3, 1, 2)
          .reshape(4 * OUT_C, 8 * IN_C).astype(jnp.bfloat16))

    ext = pl.pallas_call(
        _pass1_body,
        grid=(pl.cdiv(N, B1),),
        in_specs=[
            pl.BlockSpec(memory_space=pltpu.SMEM),
            pl.BlockSpec((IN_C, B1), lambda i: (0, i)),
            pl.BlockSpec((4, B1), lambda i: (0, i)),
            pl.BlockSpec((4 * OUT_C, 8 * IN_C), lambda i: (0, 0)),
        ],
        out_specs=pl.BlockSpec((B1, EXT), lambda i: (i, 0)),
        out_shape=jax.ShapeDtypeStruct((N, EXT), jnp.float32),
    )(mdiv, feats.T, coords.T, wt)

    mesh = plsc.VectorSubcoreMesh(core_axis_name="c", subcore_axis_name="s")
    part = pl.kernel(
        _sc_scatter_body,
        out_type=jax.ShapeDtypeStruct((2, NUM_SEG, EXT), jnp.float32),
        mesh=mesh,
        compiler_params=pltpu.CompilerParams(
            use_tc_tiling_on_sc=False, needs_layout_passes=False),
        scratch_types=[
            pltpu.VMEM((SUB, EXT), jnp.float32),
            pltpu.VMEM((SUB // CH, CH), jnp.int32),
            pltpu.VMEM_SHARED((NUM_SEG, EXT), jnp.float32),
        ],
    )(ext)

    flat, out_coords = pl.pallas_call(
        _pass2_body,
        grid=(NUM_SEG // S2,),
        in_specs=[pl.BlockSpec((2, S2, EXT), lambda i: (0, i, 0))],
        out_specs=[
            pl.BlockSpec((S2, OUT_C), lambda i: (i, 0)),
            pl.BlockSpec((S2, 4), lambda i: (i, 0)),
        ],
        out_shape=[
            jax.ShapeDtypeStruct((NUM_SEG, OUT_C), jnp.float32),
            jax.ShapeDtypeStruct((NUM_SEG, 4), jnp.int32),
        ],
    )(part)

    return flat, out_coords
